# baseline (speedup 1.0000x reference)
import functools

import jax
import jax.numpy as jnp
from jax import lax
from jax.experimental import pallas as pl
from jax.experimental.pallas import tpu as pltpu
from jax.experimental.pallas import tpu_sc as plsc

BN = 2048
SC_CHUNK = 128


def _dist_body(x_ref, e_ref, dist_ref, ind_ref):
    xb = x_ref[...]
    e = e_ref[...]
    c = e.shape[0]
    d = jax.lax.dot_general(xb, e, (((1,), (1,)), ((), ())),
                            preferred_element_type=jnp.float32)
    dist_ref[...] = d
    m = jnp.max(d, axis=-1, keepdims=True)
    iota = jax.lax.broadcasted_iota(jnp.int32, d.shape, 1).astype(jnp.float32)
    w = jnp.where(d == m, iota, float(c))
    idx = jnp.min(w, axis=-1, keepdims=True)
    ind_ref[0, 0, :] = jnp.transpose(idx.astype(jnp.int32), (1, 0))[0]


def _make_sc_gather(n_rows, c, dim):
    info = plsc.get_sparse_core_info()
    nw = info.num_cores * info.num_subcores
    per_w = n_rows // nw
    n_chunks = per_w // SC_CHUNK
    mesh = plsc.VectorSubcoreMesh(core_axis_name="c", subcore_axis_name="s")

    @functools.partial(
        pl.kernel,
        mesh=mesh,
        out_type=jax.ShapeDtypeStruct((n_rows, dim), jnp.float32),
        scratch_types=[
            pltpu.VMEM((SC_CHUNK,), jnp.int32),
            pltpu.VMEM((SC_CHUNK, dim), jnp.float32),
            pltpu.SemaphoreType.DMA,
        ],
    )
    def sc_gather(idx_hbm, table_hbm, out_hbm, idx_v, rows_v, sem):
        wid = lax.axis_index("s") * info.num_cores + lax.axis_index("c")
        for j in range(n_chunks):
            base = wid * per_w + j * SC_CHUNK
            pltpu.sync_copy(idx_hbm.at[pl.ds(base, SC_CHUNK)], idx_v)
            pltpu.async_copy(table_hbm.at[idx_v], rows_v, sem).wait()
            pltpu.sync_copy(rows_v, out_hbm.at[pl.ds(base, SC_CHUNK)])

    return sc_gather


def kernel(x, embed):
    x = x.astype(jnp.float32)
    b, n, d = x.shape
    h, c, _ = embed.shape
    N = b * n
    xf = x.reshape(N, d)
    ef = embed.reshape(c, d)
    dist, ind3 = pl.pallas_call(
        _dist_body,
        grid=(N // BN,),
        in_specs=[
            pl.BlockSpec((BN, d), lambda i: (i, 0)),
            pl.BlockSpec((c, d), lambda i: (0, 0)),
        ],
        out_specs=[
            pl.BlockSpec((BN, c), lambda i: (i, 0)),
            pl.BlockSpec((1, 1, BN), lambda i: (i, 0, 0)),
        ],
        out_shape=[
            jax.ShapeDtypeStruct((N, c), jnp.float32),
            jax.ShapeDtypeStruct((N // BN, 1, BN), jnp.int32),
        ],
    )(xf, ef)
    ind_flat = ind3.reshape(N)
    quant = _make_sc_gather(N, c, d)(ind_flat, ef)
    quantize = quant.reshape(b, n, d)
    embed_ind = ind3.reshape(b, n)
    dist_out = dist.reshape(h, b, n, c)
    return quantize, embed_ind, dist_out
